# baseline (speedup 1.0000x reference)
import functools

import jax
import jax.numpy as jnp
from jax import lax
from jax.experimental import pallas as pl
from jax.experimental.pallas import tpu as pltpu
from jax.experimental.pallas import tpu_sc as plsc

_D = 64
_NW = 32
_CH = 512
_NBUF = 3


def _emb_call(idx, table, n):
    per_w = n // _NW
    n_chunks = per_w // _CH
    mesh = plsc.VectorSubcoreMesh(core_axis_name="c", subcore_axis_name="s")

    @functools.partial(
        pl.kernel,
        mesh=mesh,
        compiler_params=pltpu.CompilerParams(use_tc_tiling_on_sc=False),
        out_type=jax.ShapeDtypeStruct((n, _D), jnp.float32),
        scratch_types=(
            [pltpu.VMEM((_CH,), jnp.int32) for _ in range(_NBUF)]
            + [pltpu.VMEM((_CH, _D), jnp.float32) for _ in range(_NBUF)]
            + [pltpu.SemaphoreType.DMA for _ in range(2 * _NBUF)]
        ),
    )
    def _emb(table_hbm, idx_hbm, out_hbm, *bufs):
        idx_v = bufs[:_NBUF]
        rows_v = bufs[_NBUF:2 * _NBUF]
        gsem = bufs[2 * _NBUF:3 * _NBUF]
        osem = bufs[3 * _NBUF:4 * _NBUF]
        wid = lax.axis_index("s") * 2 + lax.axis_index("c")
        base = wid * per_w

        def start_gather(g, b):
            off = base + g * _CH
            pltpu.sync_copy(idx_hbm.at[pl.ds(off, _CH)], idx_v[b])
            pltpu.async_copy(table_hbm.at[idx_v[b]], rows_v[b], gsem[b])

        for b in range(_NBUF):
            start_gather(b, b)

        def body(g, carry):
            for b in range(_NBUF):
                @pl.when(g % _NBUF == b)
                def _():
                    off = base + g * _CH
                    pltpu.make_async_copy(table_hbm.at[idx_v[b]], rows_v[b],
                                          gsem[b]).wait()
                    pltpu.sync_copy(rows_v[b], out_hbm.at[pl.ds(off, _CH)])
                    @pl.when(g + _NBUF < n_chunks)
                    def _():
                        start_gather(g + _NBUF, b)
            return carry

        lax.fori_loop(0, n_chunks, body, 0)

    return _emb(table, idx)


def kernel(inputs, table):
    b, l = inputs.shape
    n = b * l
    idx = inputs.reshape(n).astype(jnp.int32)
    out = _emb_call(idx, table, n)
    return out.reshape(b, l * _D)


# speedup vs baseline: 8.6449x; 8.6449x over previous
import functools

import jax
import jax.numpy as jnp
from jax import lax
from jax.experimental import pallas as pl
from jax.experimental.pallas import tpu as pltpu
from jax.experimental.pallas import tpu_sc as plsc

_D = 64
_V = 7
_NW = 32
_CH = 512
_NBUF = 3


def _emb_call(idx, tflat, n):
    per_w = n // _NW
    n_chunks = per_w // _CH
    mesh = plsc.VectorSubcoreMesh(core_axis_name="c", subcore_axis_name="s")

    @functools.partial(
        pl.kernel,
        mesh=mesh,
        compiler_params=pltpu.CompilerParams(use_tc_tiling_on_sc=False),
        out_type=jax.ShapeDtypeStruct((n * _D,), jnp.float32),
        scratch_types=(
            [pltpu.VMEM((_V * _D,), jnp.float32)]
            + [pltpu.VMEM((_CH,), jnp.int32) for _ in range(_NBUF)]
            + [pltpu.VMEM((_CH * _D,), jnp.float32) for _ in range(_NBUF)]
            + [pltpu.SemaphoreType.DMA for _ in range(2 * _NBUF)]
        ),
    )
    def _emb(table_hbm, idx_hbm, out_hbm, tab_v, *bufs):
        idx_v = bufs[:_NBUF]
        rows_v = bufs[_NBUF:2 * _NBUF]
        isem = bufs[2 * _NBUF:3 * _NBUF]
        osem = bufs[3 * _NBUF:4 * _NBUF]
        wid = lax.axis_index("s") * 2 + lax.axis_index("c")
        base = wid * per_w

        def start_idx(g, b):
            pltpu.async_copy(idx_hbm.at[pl.ds(base + g * _CH, _CH)],
                             idx_v[b], isem[b])

        for b in range(_NBUF):
            start_idx(b, b)
        pltpu.sync_copy(table_hbm, tab_v)

        def compute(b):
            def grp_body(gi, carry):
                ivec = idx_v[b][pl.ds(gi * 16, 16)] * _D
                o0 = gi * (16 * _D)
                for j in range(16):
                    tb = ivec[j]
                    o = o0 + j * _D
                    for c in range(_D // 16):
                        rows_v[b][pl.ds(o + 16 * c, 16)] = (
                            tab_v[pl.ds(tb + 16 * c, 16)])
                return carry
            lax.fori_loop(0, _CH // 16, grp_body, 0)

        def body(g, carry):
            for b in range(_NBUF):
                @pl.when(g % _NBUF == b)
                def _():
                    off = (base + g * _CH) * _D
                    pltpu.make_async_copy(
                        idx_hbm.at[pl.ds(base + g * _CH, _CH)],
                        idx_v[b], isem[b]).wait()
                    @pl.when(g >= _NBUF)
                    def _():
                        poff = (base + (g - _NBUF) * _CH) * _D
                        pltpu.make_async_copy(
                            rows_v[b],
                            out_hbm.at[pl.ds(poff, _CH * _D)],
                            osem[b]).wait()
                    compute(b)
                    pltpu.async_copy(rows_v[b],
                                     out_hbm.at[pl.ds(off, _CH * _D)],
                                     osem[b])
                    @pl.when(g + _NBUF < n_chunks)
                    def _():
                        start_idx(g + _NBUF, b)
            return carry

        lax.fori_loop(0, n_chunks, body, 0)

        for b in range(_NBUF):
            g_last = n_chunks - _NBUF + b
            off = (base + g_last * _CH) * _D
            pltpu.make_async_copy(rows_v[g_last % _NBUF],
                                  out_hbm.at[pl.ds(off, _CH * _D)],
                                  osem[g_last % _NBUF]).wait()

    return _emb(tflat, idx)


def kernel(inputs, table):
    b, l = inputs.shape
    n = b * l
    idx = inputs.reshape(n).astype(jnp.int32)
    out = _emb_call(idx, table.reshape(_V * _D), n)
    return out.reshape(b, l * _D)


# speedup vs baseline: 16.0629x; 1.8581x over previous
import functools

import jax
import jax.numpy as jnp
from jax import lax
from jax.experimental import pallas as pl
from jax.experimental.pallas import tpu as pltpu
from jax.experimental.pallas import tpu_sc as plsc

_D = 64
_V = 7
_NW = 32
_CH = 512
_NBUF = 3


def _emb_call(idx, tflat, n):
    per_w = n // _NW
    n_chunks = per_w // _CH
    mesh = plsc.VectorSubcoreMesh(core_axis_name="c", subcore_axis_name="s")

    @functools.partial(
        pl.kernel,
        mesh=mesh,
        compiler_params=pltpu.CompilerParams(use_tc_tiling_on_sc=False),
        out_type=jax.ShapeDtypeStruct((n * _D,), jnp.float32),
        scratch_types=(
            [pltpu.VMEM((_V * _D,), jnp.float32)]
            + [pltpu.VMEM((_CH,), jnp.int32) for _ in range(_NBUF)]
            + [pltpu.VMEM((_CH * _D,), jnp.float32) for _ in range(_NBUF)]
            + [pltpu.SemaphoreType.DMA for _ in range(2 * _NBUF)]
        ),
    )
    def _emb(table_hbm, idx_hbm, out_hbm, tab_v, *bufs):
        idx_v = bufs[:_NBUF]
        rows_v = bufs[_NBUF:2 * _NBUF]
        isem = bufs[2 * _NBUF:3 * _NBUF]
        osem = bufs[3 * _NBUF:4 * _NBUF]
        wid = lax.axis_index("s") * 2 + lax.axis_index("c")
        base = wid * per_w

        def start_idx(g, b):
            pltpu.async_copy(idx_hbm.at[pl.ds(base + g * _CH, _CH)],
                             idx_v[b], isem[b])

        for b in range(_NBUF):
            start_idx(b, b)
        pltpu.sync_copy(table_hbm, tab_v)

        def compute(b):
            @plsc.parallel_loop(0, _CH // 16, 1, unroll=2)
            def grp_body(gi):
                ivec = idx_v[b][pl.ds(gi * 16, 16)] * _D
                o0 = gi * (16 * _D)
                for j in range(16):
                    tb = ivec[j]
                    o = o0 + j * _D
                    for c in range(_D // 16):
                        rows_v[b][pl.ds(o + 16 * c, 16)] = (
                            tab_v[pl.ds(tb + 16 * c, 16)])

        def body(g, carry):
            for b in range(_NBUF):
                @pl.when(g % _NBUF == b)
                def _():
                    off = (base + g * _CH) * _D
                    pltpu.make_async_copy(
                        idx_hbm.at[pl.ds(base + g * _CH, _CH)],
                        idx_v[b], isem[b]).wait()
                    @pl.when(g >= _NBUF)
                    def _():
                        poff = (base + (g - _NBUF) * _CH) * _D
                        pltpu.make_async_copy(
                            rows_v[b],
                            out_hbm.at[pl.ds(poff, _CH * _D)],
                            osem[b]).wait()
                    compute(b)
                    pltpu.async_copy(rows_v[b],
                                     out_hbm.at[pl.ds(off, _CH * _D)],
                                     osem[b])
                    @pl.when(g + _NBUF < n_chunks)
                    def _():
                        start_idx(g + _NBUF, b)
            return carry

        lax.fori_loop(0, n_chunks, body, 0)

        for b in range(_NBUF):
            g_last = n_chunks - _NBUF + b
            off = (base + g_last * _CH) * _D
            pltpu.make_async_copy(rows_v[g_last % _NBUF],
                                  out_hbm.at[pl.ds(off, _CH * _D)],
                                  osem[g_last % _NBUF]).wait()

    return _emb(tflat, idx)


def kernel(inputs, table):
    b, l = inputs.shape
    n = b * l
    idx = inputs.reshape(n).astype(jnp.int32)
    out = _emb_call(idx, table.reshape(_V * _D), n)
    return out.reshape(b, l * _D)
